# baseline (speedup 1.0000x reference)
import jax
import jax.numpy as jnp
from jax import lax
from jax.experimental import pallas as pl
from jax.experimental.pallas import tpu as pltpu
from jax.experimental.pallas import tpu_sc as plsc

_HID = 2048
_E = 8
_ADIM = 128
_SCALE = 2.0
_TB = 256
_NEG = -3.0e38


def _router_body(r_ref, w_ref, logits_ref):
    logits_ref[...] = lax.dot_general(
        r_ref[...], w_ref[...], (((1,), (1,)), ((), ())),
        preferred_element_type=jnp.float32)


def _adapter_body(x_ref, y_ref, c_ref, wd_ref, wu_ref, m_ref, out_ref):
    xb = x_ref[...].astype(jnp.bfloat16)
    h = lax.dot_general(xb, wd_ref[...], (((1,), (1,)), ((), ())),
                        preferred_element_type=jnp.float32)
    g = 0.5 * h * (1.0 + lax.erf(h * 0.7071067811865476))
    mult = lax.dot_general(c_ref[...], m_ref[...], (((1,), (0,)), ((), ())),
                           preferred_element_type=jnp.float32)
    hs = (g * mult).astype(jnp.bfloat16)
    delta = lax.dot_general(hs, wu_ref[...], (((1,), (0,)), ((), ())),
                            preferred_element_type=jnp.float32)
    out_ref[...] = y_ref[...] + delta


def _sc_topk(logits_t_flat, T):
    nc, ns = 2, 16
    nw = nc * ns
    tpw = T // nw

    def body(lt_hbm, ct_hbm, lvm, cvm):
        wid = lax.axis_index("s") * nc + lax.axis_index("c")
        base_t = wid * tpw
        for e in range(_E):
            pltpu.sync_copy(lt_hbm.at[pl.ds(e * T + base_t, tpw)],
                            lvm.at[pl.ds(e * tpw, tpw)])
        for ch in range(tpw // 16):
            off = ch * 16
            ls = [lvm[pl.ds(e * tpw + off, 16)] for e in range(_E)]
            m1 = ls[0]
            i1 = jnp.zeros((16,), jnp.int32)
            m2 = jnp.full((16,), _NEG, jnp.float32)
            i2 = jnp.full((16,), -1, jnp.int32)
            for e in range(1, _E):
                le = ls[e]
                ev = jnp.full((16,), e, jnp.int32)
                gt1 = le > m1
                gt2 = le > m2
                m2 = jnp.where(gt1, m1, jnp.where(gt2, le, m2))
                i2 = jnp.where(gt1, i1, jnp.where(gt2, ev, i2))
                m1 = jnp.where(gt1, le, m1)
                i1 = jnp.where(gt1, ev, i1)
            w1 = _SCALE / (1.0 + jnp.exp(m2 - m1))
            w2 = _SCALE - w1
            zero = jnp.zeros((16,), jnp.float32)
            for e in range(_E):
                ev = jnp.full((16,), e, jnp.int32)
                cvm[pl.ds(e * tpw + off, 16)] = (
                    jnp.where(i1 == ev, w1, zero)
                    + jnp.where(i2 == ev, w2, zero))
        for e in range(_E):
            pltpu.sync_copy(cvm.at[pl.ds(e * tpw, tpw)],
                            ct_hbm.at[pl.ds(e * T + base_t, tpw)])

    return pl.kernel(
        body,
        out_type=jax.ShapeDtypeStruct((_E * T,), jnp.float32),
        mesh=plsc.VectorSubcoreMesh(core_axis_name="c", subcore_axis_name="s"),
        scratch_types=[
            pltpu.VMEM((_E * tpw,), jnp.float32),
            pltpu.VMEM((_E * tpw,), jnp.float32),
        ],
    )(logits_t_flat)


def kernel(input_hidden_states, output_hidden_states, router_hidden_states,
           W_router, W_down, W_up):
    x = input_hidden_states.reshape(-1, _HID)
    y = output_hidden_states.reshape(-1, _HID)
    r = router_hidden_states.reshape(-1, _HID)
    T = x.shape[0]
    grid = T // _TB

    logits = pl.pallas_call(
        _router_body,
        grid=(grid,),
        in_specs=[pl.BlockSpec((_TB, _HID), lambda i: (i, 0)),
                  pl.BlockSpec((_E, _HID), lambda i: (0, 0))],
        out_specs=pl.BlockSpec((_TB, _E), lambda i: (i, 0)),
        out_shape=jax.ShapeDtypeStruct((T, _E), jnp.float32),
    )(r, W_router)

    logits_t = logits.T.reshape(-1)
    c = _sc_topk(logits_t, T).reshape(_E, T).T

    wd = W_down.reshape(_E * _ADIM, _HID).astype(jnp.bfloat16)
    wu = W_up.transpose(0, 2, 1).reshape(_E * _ADIM, _HID).astype(jnp.bfloat16)
    m = jnp.repeat(jnp.eye(_E, dtype=jnp.float32), _ADIM, axis=1)

    out = pl.pallas_call(
        _adapter_body,
        grid=(grid,),
        in_specs=[pl.BlockSpec((_TB, _HID), lambda i: (i, 0)),
                  pl.BlockSpec((_TB, _HID), lambda i: (i, 0)),
                  pl.BlockSpec((_TB, _E), lambda i: (i, 0)),
                  pl.BlockSpec((_E * _ADIM, _HID), lambda i: (0, 0)),
                  pl.BlockSpec((_E * _ADIM, _HID), lambda i: (0, 0)),
                  pl.BlockSpec((_E, _E * _ADIM), lambda i: (0, 0))],
        out_specs=pl.BlockSpec((_TB, _HID), lambda i: (i, 0)),
        out_shape=jax.ShapeDtypeStruct((T, _HID), jnp.float32),
    )(x, y, c, wd, wu, m)

    return out.reshape(output_hidden_states.shape), logits


# speedup vs baseline: 1.0049x; 1.0049x over previous
import jax
import jax.numpy as jnp
from jax import lax
from jax.experimental import pallas as pl
from jax.experimental.pallas import tpu as pltpu
from jax.experimental.pallas import tpu_sc as plsc

_HID = 2048
_E = 8
_ADIM = 128
_SCALE = 2.0
_TB = 256
_NEG = -3.0e38


def _router_body(r_ref, w_ref, logits_t_ref, logits_ref):
    logits_t_ref[...] = lax.dot_general(
        w_ref[...], r_ref[...], (((1,), (1,)), ((), ())),
        preferred_element_type=jnp.float32)
    logits_ref[...] = lax.dot_general(
        r_ref[...], w_ref[...], (((1,), (1,)), ((), ())),
        preferred_element_type=jnp.float32)


def _adapter_body(x_ref, y_ref, ct_ref, wd_ref, wu_ref, m_ref, out_ref):
    xb = x_ref[...].astype(jnp.bfloat16)
    h = lax.dot_general(xb, wd_ref[...], (((1,), (1,)), ((), ())),
                        preferred_element_type=jnp.float32)
    g = 0.5 * h * (1.0 + lax.erf(h * 0.7071067811865476))
    mult = lax.dot_general(ct_ref[...], m_ref[...], (((0,), (0,)), ((), ())),
                           preferred_element_type=jnp.float32)
    hs = (g * mult).astype(jnp.bfloat16)
    delta = lax.dot_general(hs, wu_ref[...], (((1,), (0,)), ((), ())),
                            preferred_element_type=jnp.float32)
    out_ref[...] = y_ref[...] + delta


def _sc_topk(logits_t_flat, T):
    nc, ns = 2, 16
    nw = nc * ns
    tpw = T // nw

    def body(lt_hbm, ct_hbm, lvm, cvm):
        wid = lax.axis_index("s") * nc + lax.axis_index("c")
        base_t = wid * tpw
        for e in range(_E):
            pltpu.sync_copy(lt_hbm.at[pl.ds(e * T + base_t, tpw)],
                            lvm.at[pl.ds(e * tpw, tpw)])
        for ch in range(tpw // 16):
            off = ch * 16
            ls = [lvm[pl.ds(e * tpw + off, 16)] for e in range(_E)]
            m1 = ls[0]
            i1 = jnp.zeros((16,), jnp.int32)
            m2 = jnp.full((16,), _NEG, jnp.float32)
            i2 = jnp.full((16,), -1, jnp.int32)
            for e in range(1, _E):
                le = ls[e]
                ev = jnp.full((16,), e, jnp.int32)
                gt1 = le > m1
                gt2 = le > m2
                m2 = jnp.where(gt1, m1, jnp.where(gt2, le, m2))
                i2 = jnp.where(gt1, i1, jnp.where(gt2, ev, i2))
                m1 = jnp.where(gt1, le, m1)
                i1 = jnp.where(gt1, ev, i1)
            w1 = _SCALE / (1.0 + jnp.exp(m2 - m1))
            w2 = _SCALE - w1
            zero = jnp.zeros((16,), jnp.float32)
            for e in range(_E):
                ev = jnp.full((16,), e, jnp.int32)
                cvm[pl.ds(e * tpw + off, 16)] = (
                    jnp.where(i1 == ev, w1, zero)
                    + jnp.where(i2 == ev, w2, zero))
        for e in range(_E):
            pltpu.sync_copy(cvm.at[pl.ds(e * tpw, tpw)],
                            ct_hbm.at[pl.ds(e * T + base_t, tpw)])

    return pl.kernel(
        body,
        out_type=jax.ShapeDtypeStruct((_E * T,), jnp.float32),
        mesh=plsc.VectorSubcoreMesh(core_axis_name="c", subcore_axis_name="s"),
        scratch_types=[
            pltpu.VMEM((_E * tpw,), jnp.float32),
            pltpu.VMEM((_E * tpw,), jnp.float32),
        ],
    )(logits_t_flat)


def kernel(input_hidden_states, output_hidden_states, router_hidden_states,
           W_router, W_down, W_up):
    x = input_hidden_states.reshape(-1, _HID)
    y = output_hidden_states.reshape(-1, _HID)
    r = router_hidden_states.reshape(-1, _HID)
    T = x.shape[0]
    grid = T // _TB

    logits_t, logits = pl.pallas_call(
        _router_body,
        grid=(grid,),
        in_specs=[pl.BlockSpec((_TB, _HID), lambda i: (i, 0)),
                  pl.BlockSpec((_E, _HID), lambda i: (0, 0))],
        out_specs=[pl.BlockSpec((_E, _TB), lambda i: (0, i)),
                   pl.BlockSpec((_TB, _E), lambda i: (i, 0))],
        out_shape=[jax.ShapeDtypeStruct((_E, T), jnp.float32),
                   jax.ShapeDtypeStruct((T, _E), jnp.float32)],
    )(r, W_router)

    ct = _sc_topk(logits_t.reshape(-1), T).reshape(_E, T)

    wd = W_down.reshape(_E * _ADIM, _HID).astype(jnp.bfloat16)
    wu = W_up.transpose(0, 2, 1).reshape(_E * _ADIM, _HID).astype(jnp.bfloat16)
    m = jnp.repeat(jnp.eye(_E, dtype=jnp.float32), _ADIM, axis=1)

    out = pl.pallas_call(
        _adapter_body,
        grid=(grid,),
        in_specs=[pl.BlockSpec((_TB, _HID), lambda i: (i, 0)),
                  pl.BlockSpec((_TB, _HID), lambda i: (i, 0)),
                  pl.BlockSpec((_E, _TB), lambda i: (0, i)),
                  pl.BlockSpec((_E * _ADIM, _HID), lambda i: (0, 0)),
                  pl.BlockSpec((_E * _ADIM, _HID), lambda i: (0, 0)),
                  pl.BlockSpec((_E, _E * _ADIM), lambda i: (0, 0))],
        out_specs=pl.BlockSpec((_TB, _HID), lambda i: (i, 0)),
        out_shape=jax.ShapeDtypeStruct((T, _HID), jnp.float32),
    )(x, y, ct, wd, wu, m)

    return out.reshape(output_hidden_states.shape), logits


# speedup vs baseline: 1.0564x; 1.0512x over previous
import jax
import jax.numpy as jnp
from jax import lax
from jax.experimental import pallas as pl
from jax.experimental.pallas import tpu as pltpu
from jax.experimental.pallas import tpu_sc as plsc

_HID = 2048
_E = 8
_ADIM = 128
_SCALE = 2.0
_TB = 256
_NEG = -3.0e38


def _router_body(r_ref, w_ref, logits_t_ref, logits_ref):
    logits_t_ref[...] = lax.dot_general(
        w_ref[...], r_ref[...], (((1,), (1,)), ((), ())),
        preferred_element_type=jnp.float32)
    logits_ref[...] = lax.dot_general(
        r_ref[...], w_ref[...], (((1,), (1,)), ((), ())),
        preferred_element_type=jnp.float32)


def _adapter_body(x_ref, y_ref, ct_ref, wd_ref, wu_ref, out_ref):
    xb = x_ref[...].astype(jnp.bfloat16)
    wdb = wd_ref[...].astype(jnp.bfloat16)
    h = lax.dot_general(xb, wdb, (((1,), (1,)), ((), ())),
                        preferred_element_type=jnp.float32)
    g = 0.5 * h * (1.0 + lax.erf(h * 0.7071067811865476))
    cols = lax.broadcasted_iota(jnp.int32, (_E, _E * _ADIM), 1)
    rows = lax.broadcasted_iota(jnp.int32, (_E, _E * _ADIM), 0)
    m = jnp.where(cols // _ADIM == rows, 1.0, 0.0)
    mult = lax.dot_general(ct_ref[...], m, (((0,), (0,)), ((), ())),
                           preferred_element_type=jnp.float32)
    hs = (g * mult).astype(jnp.bfloat16)
    delta = lax.dot_general(hs, wu_ref[...], (((1,), (0,)), ((), ())),
                            preferred_element_type=jnp.float32)
    out_ref[...] = y_ref[...] + delta


def _sc_topk(logits_t_flat, T):
    nc, ns = 2, 16
    nw = nc * ns
    tpw = T // nw

    def body(lt_hbm, ct_hbm, lvm, cvm):
        wid = lax.axis_index("s") * nc + lax.axis_index("c")
        base_t = wid * tpw
        for e in range(_E):
            pltpu.sync_copy(lt_hbm.at[pl.ds(e * T + base_t, tpw)],
                            lvm.at[pl.ds(e * tpw, tpw)])
        for ch in range(tpw // 16):
            off = ch * 16
            ls = [lvm[pl.ds(e * tpw + off, 16)] for e in range(_E)]
            m1 = ls[0]
            i1 = jnp.zeros((16,), jnp.int32)
            m2 = jnp.full((16,), _NEG, jnp.float32)
            i2 = jnp.full((16,), -1, jnp.int32)
            for e in range(1, _E):
                le = ls[e]
                ev = jnp.full((16,), e, jnp.int32)
                gt1 = le > m1
                gt2 = le > m2
                m2 = jnp.where(gt1, m1, jnp.where(gt2, le, m2))
                i2 = jnp.where(gt1, i1, jnp.where(gt2, ev, i2))
                m1 = jnp.where(gt1, le, m1)
                i1 = jnp.where(gt1, ev, i1)
            w1 = _SCALE / (1.0 + jnp.exp(m2 - m1))
            w2 = _SCALE - w1
            zero = jnp.zeros((16,), jnp.float32)
            for e in range(_E):
                ev = jnp.full((16,), e, jnp.int32)
                cvm[pl.ds(e * tpw + off, 16)] = (
                    jnp.where(i1 == ev, w1, zero)
                    + jnp.where(i2 == ev, w2, zero))
        for e in range(_E):
            pltpu.sync_copy(cvm.at[pl.ds(e * tpw, tpw)],
                            ct_hbm.at[pl.ds(e * T + base_t, tpw)])

    return pl.kernel(
        body,
        out_type=jax.ShapeDtypeStruct((_E * T,), jnp.float32),
        mesh=plsc.VectorSubcoreMesh(core_axis_name="c", subcore_axis_name="s"),
        scratch_types=[
            pltpu.VMEM((_E * tpw,), jnp.float32),
            pltpu.VMEM((_E * tpw,), jnp.float32),
        ],
    )(logits_t_flat)


def kernel(input_hidden_states, output_hidden_states, router_hidden_states,
           W_router, W_down, W_up):
    x = input_hidden_states.reshape(-1, _HID)
    y = output_hidden_states.reshape(-1, _HID)
    r = router_hidden_states.reshape(-1, _HID)
    T = x.shape[0]
    grid = T // _TB

    logits_t, logits = pl.pallas_call(
        _router_body,
        grid=(grid,),
        in_specs=[pl.BlockSpec((_TB, _HID), lambda i: (i, 0)),
                  pl.BlockSpec((_E, _HID), lambda i: (0, 0))],
        out_specs=[pl.BlockSpec((_E, _TB), lambda i: (0, i)),
                   pl.BlockSpec((_TB, _E), lambda i: (i, 0))],
        out_shape=[jax.ShapeDtypeStruct((_E, T), jnp.float32),
                   jax.ShapeDtypeStruct((T, _E), jnp.float32)],
    )(r, W_router)

    ct = _sc_topk(logits_t.reshape(-1), T).reshape(_E, T)

    wd = W_down.reshape(_E * _ADIM, _HID)
    wu = W_up.astype(jnp.bfloat16).transpose(0, 2, 1).reshape(_E * _ADIM, _HID)

    out = pl.pallas_call(
        _adapter_body,
        grid=(grid,),
        in_specs=[pl.BlockSpec((_TB, _HID), lambda i: (i, 0)),
                  pl.BlockSpec((_TB, _HID), lambda i: (i, 0)),
                  pl.BlockSpec((_E, _TB), lambda i: (0, i)),
                  pl.BlockSpec((_E * _ADIM, _HID), lambda i: (0, 0)),
                  pl.BlockSpec((_E * _ADIM, _HID), lambda i: (0, 0))],
        out_specs=pl.BlockSpec((_TB, _HID), lambda i: (i, 0)),
        out_shape=jax.ShapeDtypeStruct((T, _HID), jnp.float32),
    )(x, y, ct, wd, wu)

    return out.reshape(output_hidden_states.shape), logits


# speedup vs baseline: 1.1299x; 1.0696x over previous
import jax
import jax.numpy as jnp
from jax import lax
from jax.experimental import pallas as pl
from jax.experimental.pallas import tpu as pltpu
from jax.experimental.pallas import tpu_sc as plsc

_HID = 2048
_E = 8
_ADIM = 128
_SCALE = 2.0
_TB = 512
_NEG = -3.0e38


def _router_body(r_ref, w_ref, logits_t_ref, logits_ref):
    logits_t_ref[...] = lax.dot_general(
        w_ref[...], r_ref[...], (((1,), (1,)), ((), ())),
        preferred_element_type=jnp.float32)
    logits_ref[...] = lax.dot_general(
        r_ref[...], w_ref[...], (((1,), (1,)), ((), ())),
        preferred_element_type=jnp.float32)


def _adapter_body(x_ref, y_ref, ct_ref, wd_ref, wu_ref, out_ref):
    xb = x_ref[...].astype(jnp.bfloat16)
    wdb = wd_ref[...].astype(jnp.bfloat16)
    h = lax.dot_general(xb, wdb, (((1,), (1,)), ((), ())),
                        preferred_element_type=jnp.float32)
    g = 0.5 * h * (1.0 + lax.erf(h * 0.7071067811865476))
    cols = lax.broadcasted_iota(jnp.int32, (_E, _E * _ADIM), 1)
    rows = lax.broadcasted_iota(jnp.int32, (_E, _E * _ADIM), 0)
    m = jnp.where(cols // _ADIM == rows, 1.0, 0.0)
    mult = lax.dot_general(ct_ref[...], m, (((0,), (0,)), ((), ())),
                           preferred_element_type=jnp.float32)
    hs = (g * mult).astype(jnp.bfloat16)
    delta = lax.dot_general(hs, wu_ref[...], (((1,), (0,)), ((), ())),
                            preferred_element_type=jnp.float32)
    out_ref[...] = y_ref[...] + delta


def _sc_topk(logits_t_flat, T):
    nc, ns = 2, 16
    nw = nc * ns
    tpw = T // nw

    def body(lt_hbm, ct_hbm, lvm, cvm):
        wid = lax.axis_index("s") * nc + lax.axis_index("c")
        base_t = wid * tpw
        for e in range(_E):
            pltpu.sync_copy(lt_hbm.at[pl.ds(e * T + base_t, tpw)],
                            lvm.at[pl.ds(e * tpw, tpw)])
        for ch in range(tpw // 16):
            off = ch * 16
            ls = [lvm[pl.ds(e * tpw + off, 16)] for e in range(_E)]
            m1 = ls[0]
            i1 = jnp.zeros((16,), jnp.int32)
            m2 = jnp.full((16,), _NEG, jnp.float32)
            i2 = jnp.full((16,), -1, jnp.int32)
            for e in range(1, _E):
                le = ls[e]
                ev = jnp.full((16,), e, jnp.int32)
                gt1 = le > m1
                gt2 = le > m2
                m2 = jnp.where(gt1, m1, jnp.where(gt2, le, m2))
                i2 = jnp.where(gt1, i1, jnp.where(gt2, ev, i2))
                m1 = jnp.where(gt1, le, m1)
                i1 = jnp.where(gt1, ev, i1)
            w1 = _SCALE / (1.0 + jnp.exp(m2 - m1))
            w2 = _SCALE - w1
            zero = jnp.zeros((16,), jnp.float32)
            for e in range(_E):
                ev = jnp.full((16,), e, jnp.int32)
                cvm[pl.ds(e * tpw + off, 16)] = (
                    jnp.where(i1 == ev, w1, zero)
                    + jnp.where(i2 == ev, w2, zero))
        for e in range(_E):
            pltpu.sync_copy(cvm.at[pl.ds(e * tpw, tpw)],
                            ct_hbm.at[pl.ds(e * T + base_t, tpw)])

    return pl.kernel(
        body,
        out_type=jax.ShapeDtypeStruct((_E * T,), jnp.float32),
        mesh=plsc.VectorSubcoreMesh(core_axis_name="c", subcore_axis_name="s"),
        scratch_types=[
            pltpu.VMEM((_E * tpw,), jnp.float32),
            pltpu.VMEM((_E * tpw,), jnp.float32),
        ],
    )(logits_t_flat)


def kernel(input_hidden_states, output_hidden_states, router_hidden_states,
           W_router, W_down, W_up):
    x = input_hidden_states.reshape(-1, _HID)
    y = output_hidden_states.reshape(-1, _HID)
    r = router_hidden_states.reshape(-1, _HID)
    T = x.shape[0]
    grid = T // _TB

    logits_t, logits = pl.pallas_call(
        _router_body,
        grid=(grid,),
        in_specs=[pl.BlockSpec((_TB, _HID), lambda i: (i, 0)),
                  pl.BlockSpec((_E, _HID), lambda i: (0, 0))],
        out_specs=[pl.BlockSpec((_E, _TB), lambda i: (0, i)),
                   pl.BlockSpec((_TB, _E), lambda i: (i, 0))],
        out_shape=[jax.ShapeDtypeStruct((_E, T), jnp.float32),
                   jax.ShapeDtypeStruct((T, _E), jnp.float32)],
    )(r, W_router)

    ct = _sc_topk(logits_t.reshape(-1), T).reshape(_E, T)

    wd = W_down.reshape(_E * _ADIM, _HID)
    wu = W_up.astype(jnp.bfloat16).transpose(0, 2, 1).reshape(_E * _ADIM, _HID)

    out = pl.pallas_call(
        _adapter_body,
        grid=(grid,),
        in_specs=[pl.BlockSpec((_TB, _HID), lambda i: (i, 0)),
                  pl.BlockSpec((_TB, _HID), lambda i: (i, 0)),
                  pl.BlockSpec((_E, _TB), lambda i: (0, i)),
                  pl.BlockSpec((_E * _ADIM, _HID), lambda i: (0, 0)),
                  pl.BlockSpec((_E * _ADIM, _HID), lambda i: (0, 0))],
        out_specs=pl.BlockSpec((_TB, _HID), lambda i: (i, 0)),
        out_shape=jax.ShapeDtypeStruct((T, _HID), jnp.float32),
    )(x, y, ct, wd, wu)

    return out.reshape(output_hidden_states.shape), logits
